# baseline (speedup 1.0000x reference)
import functools
import math

import jax
import jax.numpy as jnp
from jax import lax
from jax.experimental import pallas as pl
from jax.experimental.pallas import tpu as pltpu
from jax.experimental.pallas import tpu_sc as plsc

D = 128
NC = 2
NS = 16
CHUNK = 128
G = 32
NBUF = 2
LAG = 1


def _chunks_per_tile(e: int) -> int:
    return G * math.ceil(e / (NS * CHUNK * G))


def _zero_accum(s, rows_per_tile, accum, gbuf):
    zero = jnp.zeros((16,), jnp.float32)

    def zrow(i, carry):
        for j in range(D // 16):
            gbuf[i, pl.ds(j * 16, 16)] = zero
        return carry

    lax.fori_loop(0, CHUNK, zrow, 0)
    base = s * rows_per_tile
    off = 0
    while off < rows_per_tile:
        n = min(CHUNK, rows_per_tile - off)
        pltpu.sync_copy(gbuf.at[pl.ds(0, n)], accum.at[pl.ds(base + off, n)])
        off += n


def _process_edges(s, nch, x_hbm, row2d, col2d, accum,
                   cidx, ridx, gbufs, gsems, ssems):
    tile_base = s * nch

    def start_gather(kk, b):
        pltpu.async_copy(x_hbm.at[cidx.at[kk]], gbufs[b], gsems[b])

    def wait_gather(b):
        pltpu.make_async_copy(x_hbm.at[cidx.at[0]], gbufs[b],
                              gsems[b]).wait()

    def start_scatter(kk, b):
        pltpu.async_copy(gbufs[b], accum.at[ridx.at[kk]], ssems[b],
                         add=True)

    def wait_scatter(b):
        pltpu.make_async_copy(gbufs[b], accum.at[ridx.at[0]],
                              ssems[b]).wait()

    def group_body(g, carry):
        base = tile_base + g * G
        pltpu.sync_copy(col2d.at[pl.ds(base, G)], cidx)
        pltpu.sync_copy(row2d.at[pl.ds(base, G)], ridx)
        for kk in range(G):
            if kk >= NBUF:
                wait_scatter(kk % NBUF)
            start_gather(kk, kk % NBUF)
            j = kk - LAG
            if j >= 0:
                wait_gather(j % NBUF)
                start_scatter(j, j % NBUF)
        for j in range(G - LAG, G):
            wait_gather(j % NBUF)
            start_scatter(j, j % NBUF)
        for b in range(NBUF):
            wait_scatter(b)
        return carry

    lax.fori_loop(0, nch // G, group_body, 0)


def _write_out(s, n_nodes, rows_per_tile, accum, out_hbm, c):
    full_tiles = n_nodes // rows_per_tile
    rem = n_nodes - full_tiles * rows_per_tile
    base = s * rows_per_tile

    @pl.when(s < full_tiles)
    def _():
        pltpu.sync_copy(
            accum.at[pl.ds(base, rows_per_tile)],
            out_hbm.at[c, pl.ds(base, rows_per_tile)],
        )

    if rem > 0:
        @pl.when(s == full_tiles)
        def _():
            pltpu.sync_copy(
                accum.at[pl.ds(base, rem)],
                out_hbm.at[c, pl.ds(base, rem)],
            )


def _make_sc_spmm(n_nodes, nch1, nch2):
    mesh = plsc.VectorSubcoreMesh(core_axis_name="c", subcore_axis_name="s")
    rows_per_tile = 8 * math.ceil(n_nodes / (NS * 8))
    acc_rows = NS * rows_per_tile

    @functools.partial(
        pl.kernel,
        out_type=jax.ShapeDtypeStruct((2, n_nodes, D), jnp.float32),
        mesh=mesh,
        scratch_types=[
            pltpu.VMEM_SHARED((acc_rows, D), jnp.float32),
            pltpu.VMEM((G, CHUNK), jnp.int32),
            pltpu.VMEM((G, CHUNK), jnp.int32),
        ] + [pltpu.VMEM((CHUNK, D), jnp.float32) for _ in range(NBUF)]
          + [pltpu.SemaphoreType.DMA for _ in range(2 * NBUF)],
        compiler_params=pltpu.CompilerParams(use_tc_tiling_on_sc=False),
    )
    def spmm_kernel(x_hbm, row1, col1, row2, col2, out_hbm,
                    accum, cidx, ridx, *rest):
        gbufs = rest[:NBUF]
        gsems = rest[NBUF:2 * NBUF]
        ssems = rest[2 * NBUF:]
        c = lax.axis_index("c")
        s = lax.axis_index("s")

        _zero_accum(s, rows_per_tile, accum, gbufs[0])
        plsc.subcore_barrier()

        @pl.when(c == 0)
        def _():
            _process_edges(s, nch1, x_hbm, row1, col1, accum,
                           cidx, ridx, gbufs, gsems, ssems)

        @pl.when(c == 1)
        def _():
            _process_edges(s, nch2, x_hbm, row2, col2, accum,
                           cidx, ridx, gbufs, gsems, ssems)

        plsc.subcore_barrier()
        _write_out(s, n_nodes, rows_per_tile, accum, out_hbm, c)

    return spmm_kernel, acc_rows


def _prep_edges(edge_index, dummy_row):
    e = edge_index.shape[1]
    nch = _chunks_per_tile(e)
    ep = nch * NS * CHUNK
    row = edge_index[0].astype(jnp.int32)
    col = edge_index[1].astype(jnp.int32)
    row = jnp.pad(row, (0, ep - e), constant_values=dummy_row)
    col = jnp.pad(col, (0, ep - e), constant_values=0)
    return row.reshape(-1, CHUNK), col.reshape(-1, CHUNK), nch


def kernel(x, edge_index, edge_index2):
    n_nodes = x.shape[0]
    rows_per_tile = 8 * math.ceil(n_nodes / (NS * 8))
    dummy = NS * rows_per_tile - 1
    row1, col1, nch1 = _prep_edges(edge_index, dummy)
    row2, col2, nch2 = _prep_edges(edge_index2, dummy)
    spmm, _ = _make_sc_spmm(n_nodes, nch1, nch2)
    out2 = spmm(x, row1, col1, row2, col2)
    return jnp.concatenate([out2[0], out2[1]], axis=1)


# speedup vs baseline: 1.4889x; 1.4889x over previous
import functools
import math

import jax
import jax.numpy as jnp
from jax import lax
from jax.experimental import pallas as pl
from jax.experimental.pallas import tpu as pltpu
from jax.experimental.pallas import tpu_sc as plsc

D = 128
DH = D // 2
NC = 2
NS = 16
CHUNK = 128
G = 32
NBUF = 5
LAG = 2


def _chunks_per_tile(e: int) -> int:
    return G * math.ceil(e / (NS * CHUNK * G))


def _zero_accum(s, rows_per_tile, half, accum, gbuf):
    zero = jnp.zeros((16,), jnp.float32)

    def zrow(i, carry):
        for j in range(DH // 16):
            gbuf[i, pl.ds(j * 16, 16)] = zero
        return carry

    lax.fori_loop(0, CHUNK, zrow, 0)
    for h in range(2):
        base = h * half + s * rows_per_tile
        off = 0
        while off < rows_per_tile:
            n = min(CHUNK, rows_per_tile - off)
            pltpu.sync_copy(gbuf.at[pl.ds(0, n)],
                            accum.at[pl.ds(base + off, n)])
            off += n


def _process_edges(s, nch, x_half, row2d, col2d, accum,
                   cidx, ridx, gbufs, gsems, ssems):
    tile_base = s * nch

    def start_gather(kk, b):
        pltpu.async_copy(x_half.at[cidx.at[kk]], gbufs[b], gsems[b])

    def wait_gather(b):
        pltpu.make_async_copy(x_half.at[cidx.at[0]], gbufs[b],
                              gsems[b]).wait()

    def start_scatter(kk, b):
        pltpu.async_copy(gbufs[b], accum.at[ridx.at[kk]], ssems[b],
                         add=True)

    def wait_scatter(b):
        pltpu.make_async_copy(gbufs[b], accum.at[ridx.at[0]],
                              ssems[b]).wait()

    def group_body(g, carry):
        base = tile_base + g * G
        pltpu.sync_copy(col2d.at[pl.ds(base, G)], cidx)
        pltpu.sync_copy(row2d.at[pl.ds(base, G)], ridx)
        for kk in range(G):
            if kk >= NBUF:
                wait_scatter(kk % NBUF)
            start_gather(kk, kk % NBUF)
            j = kk - LAG
            if j >= 0:
                wait_gather(j % NBUF)
                start_scatter(j, j % NBUF)
        for j in range(G - LAG, G):
            wait_gather(j % NBUF)
            start_scatter(j, j % NBUF)
        for b in range(min(NBUF, G)):
            wait_scatter(b)
        return carry

    lax.fori_loop(0, nch // G, group_body, 0)


def _write_out(s, n_nodes, rows_per_tile, half, accum, out_hbm, c):
    full_tiles = n_nodes // rows_per_tile
    rem = n_nodes - full_tiles * rows_per_tile

    def copies(cq):
        for h in range(2):
            acc_base = h * half + s * rows_per_tile
            out_base = s * rows_per_tile
            q = h * 2 + cq

            @pl.when(s < full_tiles)
            def _():
                pltpu.sync_copy(
                    accum.at[pl.ds(acc_base, rows_per_tile)],
                    out_hbm.at[q, pl.ds(out_base, rows_per_tile)],
                )

            if rem > 0:
                @pl.when(s == full_tiles)
                def _():
                    pltpu.sync_copy(
                        accum.at[pl.ds(acc_base, rem)],
                        out_hbm.at[q, pl.ds(out_base, rem)],
                    )

    @pl.when(c == 0)
    def _():
        copies(0)

    @pl.when(c == 1)
    def _():
        copies(1)


def _make_sc_spmm(n_nodes, nch):
    mesh = plsc.VectorSubcoreMesh(core_axis_name="c", subcore_axis_name="s")
    rows_per_tile = 8 * math.ceil(n_nodes / (NS * 8))
    half = NS * rows_per_tile

    @functools.partial(
        pl.kernel,
        out_type=jax.ShapeDtypeStruct((4, n_nodes, DH), jnp.float32),
        mesh=mesh,
        scratch_types=[
            pltpu.VMEM_SHARED((2 * half, DH), jnp.float32),
            pltpu.VMEM((G, CHUNK), jnp.int32),
            pltpu.VMEM((G, CHUNK), jnp.int32),
        ] + [pltpu.VMEM((CHUNK, DH), jnp.float32) for _ in range(NBUF)]
          + [pltpu.SemaphoreType.DMA for _ in range(2 * NBUF)],
        compiler_params=pltpu.CompilerParams(use_tc_tiling_on_sc=False),
    )
    def spmm_kernel(x3_hbm, row2d, col2d, out_hbm, accum, cidx, ridx, *rest):
        gbufs = rest[:NBUF]
        gsems = rest[NBUF:2 * NBUF]
        ssems = rest[2 * NBUF:]
        c = lax.axis_index("c")
        s = lax.axis_index("s")

        _zero_accum(s, rows_per_tile, half, accum, gbufs[0])
        plsc.subcore_barrier()

        _process_edges(s, nch, x3_hbm.at[c], row2d, col2d, accum,
                       cidx, ridx, gbufs, gsems, ssems)

        plsc.subcore_barrier()
        _write_out(s, n_nodes, rows_per_tile, half, accum, out_hbm, c)

    return spmm_kernel, half


def _prep_edges(edge_index, row_offset, dummy_row):
    e = edge_index.shape[1]
    nch = _chunks_per_tile(e)
    ep = nch * NS * CHUNK
    row = edge_index[0].astype(jnp.int32) + row_offset
    col = edge_index[1].astype(jnp.int32)
    row = jnp.pad(row, (0, ep - e), constant_values=dummy_row)
    col = jnp.pad(col, (0, ep - e), constant_values=0)
    return (row.reshape(NS, nch, CHUNK), col.reshape(NS, nch, CHUNK), nch)


def kernel(x, edge_index, edge_index2):
    n_nodes = x.shape[0]
    rows_per_tile = 8 * math.ceil(n_nodes / (NS * 8))
    half = NS * rows_per_tile
    r1, c1, nch1 = _prep_edges(edge_index, 0, half - 1)
    r2, c2, nch2 = _prep_edges(edge_index2, half, 2 * half - 1)
    row2d = jnp.concatenate([r1, r2], axis=1).reshape(-1, CHUNK)
    col2d = jnp.concatenate([c1, c2], axis=1).reshape(-1, CHUNK)
    x3 = jnp.stack([x[:, :DH], x[:, DH:]])
    spmm, _ = _make_sc_spmm(n_nodes, nch1 + nch2)
    out4 = spmm(x3, row2d, col2d)
    return jnp.concatenate([out4[0], out4[1], out4[2], out4[3]], axis=1)


# speedup vs baseline: 2.3243x; 1.5611x over previous
import functools
import math

import numpy as np

import jax
import jax.numpy as jnp
from jax import lax
from jax.experimental import pallas as pl
from jax.experimental.pallas import tpu as pltpu
from jax.experimental.pallas import tpu_sc as plsc

D = 128
DH = D // 2
NC = 2
NS = 16
CHUNK = 128
G = 32
NBUF = 6
LAG = 2

_HALF_PERM = np.argsort(
    np.array(
        list(range(0, 32, 2)) + list(range(1, 32, 2))
        + list(range(32, 64, 2)) + list(range(33, 64, 2))
    )
)
_COL_PERM = np.concatenate([_HALF_PERM, _HALF_PERM + DH])


def _chunks_per_tile(e: int) -> int:
    return G * math.ceil(e / (NS * CHUNK * G))


def _zero_accum(s, rows_per_tile, half, accum, gbuf):
    zero = jnp.zeros((32,), jnp.bfloat16)

    def zrow(i, carry):
        for j in range(DH // 32):
            gbuf[i, pl.ds(j * 32, 32)] = zero
        return carry

    lax.fori_loop(0, CHUNK, zrow, 0)
    for h in range(2):
        base = h * half + s * rows_per_tile
        off = 0
        while off < rows_per_tile:
            n = min(CHUNK, rows_per_tile - off)
            pltpu.sync_copy(gbuf.at[pl.ds(0, n)],
                            accum.at[pl.ds(base + off, n)])
            off += n


def _process_edges(s, nch, x_half, row2d, col2d, accum,
                   cidx, ridx, gbufs, gsems, ssems):
    tile_base = s * nch

    def start_gather(kk, b):
        pltpu.async_copy(x_half.at[cidx.at[kk]], gbufs[b], gsems[b])

    def wait_gather(b):
        pltpu.make_async_copy(x_half.at[cidx.at[0]], gbufs[b],
                              gsems[b]).wait()

    def start_scatter(kk, b):
        pltpu.async_copy(gbufs[b], accum.at[ridx.at[kk]], ssems[b],
                         add=True)

    def wait_scatter(b):
        pltpu.make_async_copy(gbufs[b], accum.at[ridx.at[0]],
                              ssems[b]).wait()

    def group_body(g, carry):
        base = tile_base + g * G
        pltpu.sync_copy(col2d.at[pl.ds(base, G)], cidx)
        pltpu.sync_copy(row2d.at[pl.ds(base, G)], ridx)
        for kk in range(G):
            if kk >= NBUF:
                wait_scatter(kk % NBUF)
            start_gather(kk, kk % NBUF)
            j = kk - LAG
            if j >= 0:
                wait_gather(j % NBUF)
                start_scatter(j, j % NBUF)
        for j in range(G - LAG, G):
            wait_gather(j % NBUF)
            start_scatter(j, j % NBUF)
        for b in range(min(NBUF, G)):
            wait_scatter(b)
        return carry

    lax.fori_loop(0, nch // G, group_body, 0)


def _widen_rows(nrows, cbuf, fbuf):
    hi16 = jnp.uint32(0xFFFF0000)

    def crow(i, carry):
        for j in range(DH // 32):
            v = cbuf[i, pl.ds(j * 32, 32)]
            w = plsc.bitcast(v, jnp.uint32)
            ev = plsc.bitcast(w << 16, jnp.float32)
            od = plsc.bitcast(w & hi16, jnp.float32)
            fbuf[i, pl.ds(j * 32, 16)] = ev
            fbuf[i, pl.ds(j * 32 + 16, 16)] = od
        return carry

    lax.fori_loop(0, nrows, crow, 0)


def _write_out(s, n_nodes, rows_per_tile, half, accum, out_hbm, c,
               cbuf, fbuf):
    full_tiles = n_nodes // rows_per_tile
    rem = n_nodes - full_tiles * rows_per_tile

    def copies(cq):
        for h in range(2):
            q = h * 2 + cq

            def emit(nrows):
                acc_base = h * half + s * rows_per_tile
                out_base = s * rows_per_tile
                off = 0
                while off < nrows:
                    n = min(CHUNK, nrows - off)
                    pltpu.sync_copy(accum.at[pl.ds(acc_base + off, n)],
                                    cbuf.at[pl.ds(0, n)])
                    _widen_rows(n, cbuf, fbuf)
                    pltpu.sync_copy(fbuf.at[pl.ds(0, n)],
                                    out_hbm.at[q, pl.ds(out_base + off, n)])
                    off += n

            @pl.when(s < full_tiles)
            def _():
                emit(rows_per_tile)

            if rem > 0:
                @pl.when(s == full_tiles)
                def _():
                    emit(rem)

    @pl.when(c == 0)
    def _():
        copies(0)

    @pl.when(c == 1)
    def _():
        copies(1)


def _make_sc_spmm(n_nodes, nch):
    mesh = plsc.VectorSubcoreMesh(core_axis_name="c", subcore_axis_name="s")
    rows_per_tile = 8 * math.ceil(n_nodes / (NS * 8))
    half = NS * rows_per_tile

    @functools.partial(
        pl.kernel,
        out_type=jax.ShapeDtypeStruct((4, n_nodes, DH), jnp.float32),
        mesh=mesh,
        scratch_types=[
            pltpu.VMEM_SHARED((2 * half, DH), jnp.bfloat16),
            pltpu.VMEM((G, CHUNK), jnp.int32),
            pltpu.VMEM((G, CHUNK), jnp.int32),
            pltpu.VMEM((CHUNK, DH), jnp.float32),
        ] + [pltpu.VMEM((CHUNK, DH), jnp.bfloat16) for _ in range(NBUF)]
          + [pltpu.SemaphoreType.DMA for _ in range(2 * NBUF)],
        compiler_params=pltpu.CompilerParams(use_tc_tiling_on_sc=False,
                                             needs_layout_passes=False),
    )
    def spmm_kernel(x3_hbm, row2d, col2d, out_hbm,
                    accum, cidx, ridx, fbuf, *rest):
        gbufs = rest[:NBUF]
        gsems = rest[NBUF:2 * NBUF]
        ssems = rest[2 * NBUF:]
        c = lax.axis_index("c")
        s = lax.axis_index("s")

        _zero_accum(s, rows_per_tile, half, accum, gbufs[0])
        plsc.subcore_barrier()

        _process_edges(s, nch, x3_hbm.at[c], row2d, col2d, accum,
                       cidx, ridx, gbufs, gsems, ssems)

        plsc.subcore_barrier()
        _write_out(s, n_nodes, rows_per_tile, half, accum, out_hbm, c,
                   gbufs[0], fbuf)

    return spmm_kernel, half


def _prep_edges(edge_index, row_offset, dummy_row):
    e = edge_index.shape[1]
    nch = _chunks_per_tile(e)
    ep = nch * NS * CHUNK
    row = edge_index[0].astype(jnp.int32) + row_offset
    col = edge_index[1].astype(jnp.int32)
    row = jnp.pad(row, (0, ep - e), constant_values=dummy_row)
    col = jnp.pad(col, (0, ep - e), constant_values=0)
    return (row.reshape(NS, nch, CHUNK), col.reshape(NS, nch, CHUNK), nch)


def kernel(x, edge_index, edge_index2):
    n_nodes = x.shape[0]
    rows_per_tile = 8 * math.ceil(n_nodes / (NS * 8))
    half = NS * rows_per_tile
    r1, c1, nch1 = _prep_edges(edge_index, 0, half - 1)
    r2, c2, nch2 = _prep_edges(edge_index2, half, 2 * half - 1)
    row2d = jnp.concatenate([r1, r2], axis=1).reshape(-1, CHUNK)
    col2d = jnp.concatenate([c1, c2], axis=1).reshape(-1, CHUNK)
    xp = x.astype(jnp.bfloat16)[:, _COL_PERM]
    x3 = jnp.stack([xp[:, :DH], xp[:, DH:]])
    spmm, _ = _make_sc_spmm(n_nodes, nch1 + nch2)
    out4 = spmm(x3, row2d, col2d)
    return jnp.concatenate([out4[0], out4[1], out4[2], out4[3]], axis=1)
